# baseline (speedup 1.0000x reference)
import functools

import jax
import jax.numpy as jnp
from jax import lax
from jax.experimental import pallas as pl
from jax.experimental.pallas import tpu as pltpu
from jax.experimental.pallas import tpu_sc as plsc

D = 64
C = 10
NNEG = 20
ROWS = C + C * NNEG
RP = 216
BUF_ROWS = 224
NBLK = BUF_ROWS // 16
CH0, CH1 = 112, 104
NC, NS = 2, 16
NW = NC * NS


def _log1p_series(u):
    t = u / (2.0 + u)
    t2 = t * t
    p = 1.0 / 9.0 + t2 * (1.0 / 11.0)
    p = 1.0 / 7.0 + t2 * p
    p = 1.0 / 5.0 + t2 * p
    p = 1.0 / 3.0 + t2 * p
    return 2.0 * t * (1.0 + t2 * p)


def _body(bpw, ovec_hbm, idx_hbm, iw_hbm, ivec_hbm, out_hbm,
          idx_v, iw_v, ivb, buf0, buf1, acc_v, semi, sem0, sem1):
    wid = lax.axis_index("c") * NS + lax.axis_index("s")
    base = wid * bpw
    lane = lax.iota(jnp.int32, 16)

    pltpu.sync_copy(idx_hbm.at[pl.ds(base * RP, bpw * RP)], idx_v)
    pltpu.sync_copy(iw_hbm.at[pl.ds(base, bpw)], iw_v)
    pltpu.async_copy(ivec_hbm.at[iw_v], ivb, semi).wait()

    zero16 = jnp.zeros((16,), jnp.float32)
    for buf in (buf0, buf1):
        for r in range(RP, BUF_ROWS):
            for c4 in range(D // 16):
                buf[r, pl.ds(c4 * 16, 16)] = zero16

    bufs = (buf0, buf1)
    sems = (sem0, sem1)

    def issue(j, b):
        i0 = b * RP
        pltpu.async_copy(ovec_hbm.at[idx_v.at[pl.ds(i0, CH0)]],
                         bufs[j].at[pl.ds(0, CH0)], sems[j])
        pltpu.async_copy(ovec_hbm.at[idx_v.at[pl.ds(i0 + CH0, CH1)]],
                         bufs[j].at[pl.ds(CH0, CH1)], sems[j])

    def drain(j, b):
        i0 = b * RP
        pltpu.make_async_copy(ovec_hbm.at[idx_v.at[pl.ds(i0, CH0)]],
                              bufs[j].at[pl.ds(0, CH0)], sems[j]).wait()
        pltpu.make_async_copy(ovec_hbm.at[idx_v.at[pl.ds(i0 + CH0, CH1)]],
                              bufs[j].at[pl.ds(CH0, CH1)], sems[j]).wait()

    def compute(j, b, acc):
        buf = bufs[j]
        ivs = [ivb[b, pl.ds(g * 16, 16)] for g in range(D // 16)]

        def kstep(k, acc):
            rowv = lane + k * 16
            s = zero16
            for g in range(D // 16):
                ivg = ivs[g]
                for dd in range(16):
                    colv = jnp.full((16,), g * 16 + dd, jnp.int32)
                    s = s + plsc.load_gather(buf, [rowv, colv]) * ivg[dd]
            z = jnp.where(rowv < C, -s, s)
            w = jnp.where(rowv < ROWS, 0.1, 0.0)
            u = jnp.exp(-jnp.abs(z))
            return acc + w * (jnp.maximum(z, 0.0) + _log1p_series(u))

        return lax.fori_loop(0, NBLK, kstep, acc)

    issue(0, 0)
    issue(1, 1)

    def outer(o, acc):
        for j in range(2):
            b = o * 2 + j
            drain(j, b)
            acc = compute(j, b, acc)

            @pl.when(b + 2 < bpw)
            def _():
                issue(j, b + 2)
        return acc

    acc = lax.fori_loop(0, bpw // 2, outer, zero16)
    acc_v[...] = acc * (1.0 / float(bpw * NW))
    pltpu.sync_copy(acc_v, out_hbm.at[wid])


def kernel(iword, owords, nwords, ivec_w, ovec_w):
    b = iword.shape[0]
    bpw = b // NW
    idx_all = jnp.concatenate(
        [owords.astype(jnp.int32), nwords.astype(jnp.int32),
         jnp.zeros((b, RP - ROWS), jnp.int32)], axis=1).reshape(-1)
    mesh = plsc.VectorSubcoreMesh(core_axis_name="c", subcore_axis_name="s")
    run = pl.kernel(
        functools.partial(_body, bpw),
        out_type=jax.ShapeDtypeStruct((NW, 16), jnp.float32),
        mesh=mesh,
        compiler_params=pltpu.CompilerParams(needs_layout_passes=False,
                                             use_tc_tiling_on_sc=False),
        scratch_types=[
            pltpu.VMEM((bpw * RP,), jnp.int32),
            pltpu.VMEM((bpw,), jnp.int32),
            pltpu.VMEM((bpw, D), jnp.float32),
            pltpu.VMEM((BUF_ROWS, D), jnp.float32),
            pltpu.VMEM((BUF_ROWS, D), jnp.float32),
            pltpu.VMEM((16,), jnp.float32),
            pltpu.SemaphoreType.DMA,
            pltpu.SemaphoreType.DMA,
            pltpu.SemaphoreType.DMA,
        ],
    )
    partials = run(ovec_w, idx_all, iword.astype(jnp.int32), ivec_w)
    return jnp.sum(partials)


# speedup vs baseline: 1.1452x; 1.1452x over previous
import functools

import jax
import jax.numpy as jnp
from jax import lax
from jax.experimental import pallas as pl
from jax.experimental.pallas import tpu as pltpu
from jax.experimental.pallas import tpu_sc as plsc

D = 64
C = 10
NNEG = 20
ROWS = C + C * NNEG
RP = 216
BUF_ROWS = 224
NBLK = BUF_ROWS // 16
CH0, CH1 = 112, 104
NC, NS = 2, 16
NW = NC * NS


def _perm(v, idx):
    return lax.gather(
        v, idx[:, None],
        lax.GatherDimensionNumbers(offset_dims=(), collapsed_slice_dims=(0,),
                                   start_index_map=(0,)),
        (1,), mode=lax.GatherScatterMode.PROMISE_IN_BOUNDS)


def _log1p_series(u):
    t = u / (2.0 + u)
    t2 = t * t
    p = 1.0 / 9.0 + t2 * (1.0 / 11.0)
    p = 1.0 / 7.0 + t2 * p
    p = 1.0 / 5.0 + t2 * p
    p = 1.0 / 3.0 + t2 * p
    return 2.0 * t * (1.0 + t2 * p)


def _body(bpw, ovec_hbm, idx_hbm, iw_hbm, ivec_hbm, out_hbm,
          idx_v, iw_v, ivb, buf0, buf1, acc_v, semi, sem0, sem1):
    wid = lax.axis_index("c") * NS + lax.axis_index("s")
    base = wid * bpw
    lane = lax.iota(jnp.int32, 16)

    pltpu.sync_copy(idx_hbm.at[pl.ds(base * RP, bpw * RP)], idx_v)
    pltpu.sync_copy(iw_hbm.at[pl.ds(base, bpw)], iw_v)
    pltpu.async_copy(ivec_hbm.at[iw_v], ivb, semi).wait()

    zero16 = jnp.zeros((16,), jnp.float32)
    for buf in (buf0, buf1):
        for r in range(RP, BUF_ROWS):
            for c4 in range(D // 16):
                buf[r, pl.ds(c4 * 16, 16)] = zero16

    bufs = (buf0, buf1)
    sems = (sem0, sem1)

    def issue(j, b):
        i0 = b * RP
        pltpu.async_copy(ovec_hbm.at[idx_v.at[pl.ds(i0, CH0)]],
                         bufs[j].at[pl.ds(0, CH0)], sems[j])
        pltpu.async_copy(ovec_hbm.at[idx_v.at[pl.ds(i0 + CH0, CH1)]],
                         bufs[j].at[pl.ds(CH0, CH1)], sems[j])

    def drain(j, b):
        i0 = b * RP
        pltpu.make_async_copy(ovec_hbm.at[idx_v.at[pl.ds(i0, CH0)]],
                              bufs[j].at[pl.ds(0, CH0)], sems[j]).wait()
        pltpu.make_async_copy(ovec_hbm.at[idx_v.at[pl.ds(i0 + CH0, CH1)]],
                              bufs[j].at[pl.ds(CH0, CH1)], sems[j]).wait()

    xor_idx = [jnp.asarray(jnp.arange(16, dtype=jnp.int32) ^ (1 << lv))
               for lv in range(4)]

    def compute(j, b, acc):
        buf = bufs[j]
        ivs = [ivb[b, pl.ds(g * 16, 16)] for g in range(D // 16)]

        def kstep(k, acc):
            rowv = lane + k * 16
            vecs = []
            for r in range(16):
                row = k * 16 + r
                p = buf[row, pl.ds(0, 16)] * ivs[0]
                for g in range(1, D // 16):
                    p = p + buf[row, pl.ds(g * 16, 16)] * ivs[g]
                vecs.append(p)
            for lv in range(4):
                m = 1 << lv
                sel = (lane & m) == 0
                nxt = []
                for q in range(len(vecs) // 2):
                    a = vecs[2 * q]
                    bb = vecs[2 * q + 1]
                    a = a + _perm(a, xor_idx[lv])
                    bb = bb + _perm(bb, xor_idx[lv])
                    nxt.append(jnp.where(sel, a, bb))
                vecs = nxt
            s = vecs[0]
            z = jnp.where(rowv < C, -s, s)
            w = jnp.where(rowv < ROWS, 0.1, 0.0)
            u = jnp.exp(-jnp.abs(z))
            return acc + w * (jnp.maximum(z, 0.0) + _log1p_series(u))

        return lax.fori_loop(0, NBLK, kstep, acc)

    issue(0, 0)
    issue(1, 1)

    def outer(o, acc):
        for j in range(2):
            b = o * 2 + j
            drain(j, b)
            acc = compute(j, b, acc)

            @pl.when(b + 2 < bpw)
            def _():
                issue(j, b + 2)
        return acc

    acc = lax.fori_loop(0, bpw // 2, outer, zero16)
    acc_v[...] = acc * (1.0 / float(bpw * NW))
    pltpu.sync_copy(acc_v, out_hbm.at[wid])


def kernel(iword, owords, nwords, ivec_w, ovec_w):
    b = iword.shape[0]
    bpw = b // NW
    idx_all = jnp.concatenate(
        [owords.astype(jnp.int32), nwords.astype(jnp.int32),
         jnp.zeros((b, RP - ROWS), jnp.int32)], axis=1).reshape(-1)
    mesh = plsc.VectorSubcoreMesh(core_axis_name="c", subcore_axis_name="s")
    run = pl.kernel(
        functools.partial(_body, bpw),
        out_type=jax.ShapeDtypeStruct((NW, 16), jnp.float32),
        mesh=mesh,
        compiler_params=pltpu.CompilerParams(needs_layout_passes=False,
                                             use_tc_tiling_on_sc=False),
        scratch_types=[
            pltpu.VMEM((bpw * RP,), jnp.int32),
            pltpu.VMEM((bpw,), jnp.int32),
            pltpu.VMEM((bpw, D), jnp.float32),
            pltpu.VMEM((BUF_ROWS, D), jnp.float32),
            pltpu.VMEM((BUF_ROWS, D), jnp.float32),
            pltpu.VMEM((16,), jnp.float32),
            pltpu.SemaphoreType.DMA,
            pltpu.SemaphoreType.DMA,
            pltpu.SemaphoreType.DMA,
        ],
    )
    partials = run(ovec_w, idx_all, iword.astype(jnp.int32), ivec_w)
    return jnp.sum(partials)
